# baseline (speedup 1.0000x reference)
import jax
import jax.numpy as jnp
from jax.experimental import pallas as pl
from jax.experimental.pallas import tpu as pltpu

N = 8192
BM = 256
NBLK = N // BM
D = 128


def _fused_gnn(xp_ref, adj_ref, w1_ref, w2_ref, w3_ref, wd1_ref, wd2_ref,
               wd3_ref, bias_ref, out_ref, ya, yb, g_acc):
    l = pl.program_id(0)
    i = pl.program_id(1)

    @pl.when(jnp.logical_and(l == 0, i == 0))
    def _init():
        ya[...] = jnp.dot(xp_ref[...], w1_ref[...],
                          preferred_element_type=jnp.float32)
        g_acc[...] = jnp.zeros_like(g_acc)

    a_blk = adj_ref[...]

    @pl.when(l == 0)
    def _layer0():
        h = jnp.maximum(
            jnp.dot(a_blk, ya[...], preferred_element_type=jnp.float32)
            + bias_ref[0, :][None, :], 0.0)
        yb[pl.ds(i * BM, BM), :] = jnp.dot(
            h, w2_ref[...], preferred_element_type=jnp.float32)

    @pl.when(l == 1)
    def _layer1():
        h = jnp.maximum(
            jnp.dot(a_blk, yb[...], preferred_element_type=jnp.float32)
            + bias_ref[1, :][None, :], 0.0)
        ya[pl.ds(i * BM, BM), :] = jnp.dot(
            h, w3_ref[...], preferred_element_type=jnp.float32)

    @pl.when(l == 2)
    def _layer2():
        h = jnp.maximum(
            jnp.dot(a_blk, ya[...], preferred_element_type=jnp.float32)
            + bias_ref[2, :][None, :], 0.0)
        g_acc[...] += jnp.sum(h, axis=0, keepdims=True)

    @pl.when(jnp.logical_and(l == 2, i == NBLK - 1))
    def _head():
        g = g_acc[...]
        norm = jnp.maximum(jnp.sqrt(jnp.sum(g * g)), 1e-12)
        gn = g / norm
        d1 = jnp.maximum(
            jnp.dot(gn, wd1_ref[...], preferred_element_type=jnp.float32)
            + bias_ref[3, :][None, :], 0.0)
        d2 = jnp.maximum(
            jnp.dot(d1, wd2_ref[...], preferred_element_type=jnp.float32)
            + bias_ref[4, :][None, :], 0.0)
        d3 = (jnp.dot(d2, wd3_ref[...], preferred_element_type=jnp.float32)
              + bias_ref[5, :][None, :])
        out_ref[...] = jnp.broadcast_to(d3, out_ref.shape)


def _pad2(w, rows, cols):
    return jnp.pad(w, ((0, rows - w.shape[0]), (0, cols - w.shape[1])))


def kernel(x, adj, W1, b1, W2, b2, W3, b3, Wd1, bd1, Wd2, bd2, Wd3, bd3):
    xp = jnp.pad(x, ((0, 0), (0, D - x.shape[1])))
    w1 = _pad2(W1, D, D)
    w2 = _pad2(W2, D, D)
    w3 = _pad2(W3, D, D)
    wd1 = _pad2(Wd1, D, D)
    wd2 = _pad2(Wd2, D, D)
    wd3 = _pad2(Wd3, D, D)
    bias = jnp.zeros((8, D), jnp.float32)
    bias = bias.at[0, :16].set(b1).at[1, :32].set(b2).at[2, :64].set(b3)
    bias = bias.at[3, :128].set(bd1).at[4, :64].set(bd2).at[5, :1].set(bd3)

    full = lambda shape: pl.BlockSpec(shape, lambda l, i: (0,) * len(shape))
    out = pl.pallas_call(
        _fused_gnn,
        grid=(3, NBLK),
        in_specs=[
            full((N, D)),
            pl.BlockSpec((BM, N), lambda l, i: (i, 0)),
            full((D, D)), full((D, D)), full((D, D)),
            full((D, D)), full((D, D)), full((D, D)),
            full((8, D)),
        ],
        out_specs=pl.BlockSpec((8, D), lambda l, i: (0, 0)),
        out_shape=jax.ShapeDtypeStruct((8, D), jnp.float32),
        scratch_shapes=[
            pltpu.VMEM((N, D), jnp.float32),
            pltpu.VMEM((N, D), jnp.float32),
            pltpu.VMEM((1, D), jnp.float32),
        ],
        compiler_params=pltpu.CompilerParams(
            dimension_semantics=("arbitrary", "arbitrary")),
    )(xp, adj, w1, w2, w3, wd1, wd2, wd3, bias)
    return out[0, 0:1]


# speedup vs baseline: 1.0235x; 1.0235x over previous
import jax
import jax.numpy as jnp
from jax.experimental import pallas as pl
from jax.experimental.pallas import tpu as pltpu

N = 8192
BM = 256
NBLK = N // BM
D = 128


def _fused_gnn(xp_ref, adj_ref, w1_ref, w2_ref, w3_ref, wd1_ref, wd2_ref,
               wd3_ref, bias_ref, out_ref, ya, yb, g_acc):
    l = pl.program_id(0)
    i = pl.program_id(1)

    @pl.when(jnp.logical_and(l == 0, i == 0))
    def _init():
        ya[...] = jnp.dot(xp_ref[...], w1_ref[...],
                          preferred_element_type=jnp.float32)
        g_acc[...] = jnp.zeros_like(g_acc)

    a_blk = adj_ref[...].astype(jnp.bfloat16)

    @pl.when(l == 0)
    def _layer0():
        h = jnp.maximum(
            jnp.dot(a_blk, ya[...].astype(jnp.bfloat16),
                    preferred_element_type=jnp.float32)
            + bias_ref[0, :][None, :], 0.0)
        yb[pl.ds(i * BM, BM), :] = jnp.dot(
            h, w2_ref[...], preferred_element_type=jnp.float32)

    @pl.when(l == 1)
    def _layer1():
        h = jnp.maximum(
            jnp.dot(a_blk, yb[...].astype(jnp.bfloat16),
                    preferred_element_type=jnp.float32)
            + bias_ref[1, :][None, :], 0.0)
        ya[pl.ds(i * BM, BM), :] = jnp.dot(
            h, w3_ref[...], preferred_element_type=jnp.float32)

    @pl.when(l == 2)
    def _layer2():
        h = jnp.maximum(
            jnp.dot(a_blk, ya[...].astype(jnp.bfloat16),
                    preferred_element_type=jnp.float32)
            + bias_ref[2, :][None, :], 0.0)
        g_acc[...] += jnp.sum(h, axis=0, keepdims=True)

    @pl.when(jnp.logical_and(l == 2, i == NBLK - 1))
    def _head():
        g = g_acc[...]
        norm = jnp.maximum(jnp.sqrt(jnp.sum(g * g)), 1e-12)
        gn = g / norm
        d1 = jnp.maximum(
            jnp.dot(gn, wd1_ref[...], preferred_element_type=jnp.float32)
            + bias_ref[3, :][None, :], 0.0)
        d2 = jnp.maximum(
            jnp.dot(d1, wd2_ref[...], preferred_element_type=jnp.float32)
            + bias_ref[4, :][None, :], 0.0)
        d3 = (jnp.dot(d2, wd3_ref[...], preferred_element_type=jnp.float32)
              + bias_ref[5, :][None, :])
        out_ref[...] = jnp.broadcast_to(d3, out_ref.shape)


def _pad2(w, rows, cols):
    return jnp.pad(w, ((0, rows - w.shape[0]), (0, cols - w.shape[1])))


def kernel(x, adj, W1, b1, W2, b2, W3, b3, Wd1, bd1, Wd2, bd2, Wd3, bd3):
    xp = jnp.pad(x, ((0, 0), (0, D - x.shape[1])))
    w1 = _pad2(W1, D, D)
    w2 = _pad2(W2, D, D)
    w3 = _pad2(W3, D, D)
    wd1 = _pad2(Wd1, D, D)
    wd2 = _pad2(Wd2, D, D)
    wd3 = _pad2(Wd3, D, D)
    bias = jnp.zeros((8, D), jnp.float32)
    bias = bias.at[0, :16].set(b1).at[1, :32].set(b2).at[2, :64].set(b3)
    bias = bias.at[3, :128].set(bd1).at[4, :64].set(bd2).at[5, :1].set(bd3)

    full = lambda shape: pl.BlockSpec(shape, lambda l, i: (0,) * len(shape))
    out = pl.pallas_call(
        _fused_gnn,
        grid=(3, NBLK),
        in_specs=[
            full((N, D)),
            pl.BlockSpec((BM, N), lambda l, i: (i, 0)),
            full((D, D)), full((D, D)), full((D, D)),
            full((D, D)), full((D, D)), full((D, D)),
            full((8, D)),
        ],
        out_specs=pl.BlockSpec((8, D), lambda l, i: (0, 0)),
        out_shape=jax.ShapeDtypeStruct((8, D), jnp.float32),
        scratch_shapes=[
            pltpu.VMEM((N, D), jnp.float32),
            pltpu.VMEM((N, D), jnp.float32),
            pltpu.VMEM((1, D), jnp.float32),
        ],
        compiler_params=pltpu.CompilerParams(
            dimension_semantics=("arbitrary", "arbitrary")),
    )(xp, adj, w1, w2, w3, wd1, wd2, wd3, bias)
    return out[0, 0:1]
